# baseline (speedup 1.0000x reference)
import functools

import jax
import jax.numpy as jnp
from jax.experimental import pallas as pl
from jax.experimental.pallas import tpu as pltpu

B, L, D = 4, 2048, 1024
KD, VD, S, H, TOPK = 64, 256, 100000, 256, 8

SC_CHUNK = 8192
NSC = (S + SC_CHUNK - 1) // SC_CHUNK
S_PAD = NSC * SC_CHUNK
SROWS = S_PAD // 128
LT = 256
NLT = L // LT
LTQ = 512
NLTQ = L // LTQ


def _qs_body(x_ref, wk_ref, bk_ref, keys_ref, s_ref, acc_ref, q_ref):
    j = pl.program_id(0)

    @pl.when(j < NLTQ)
    def _():
        part = jnp.sum(x_ref[...], axis=1)

        @pl.when(j == 0)
        def _():
            acc_ref[...] = part

        @pl.when(j > 0)
        def _():
            acc_ref[...] = acc_ref[...] + part

        @pl.when(j == NLTQ - 1)
        def _():
            xbar = acc_ref[...] * (1.0 / L)
            q_ref[0:B] = (
                jnp.dot(xbar, wk_ref[...], preferred_element_type=jnp.float32)
                + bk_ref[...]
            )

    @pl.when(j >= NLTQ)
    def _():
        c = j - NLTQ
        s = jax.lax.dot_general(
            q_ref[0:B], keys_ref[...], (((1,), (1,)), ((), ())),
            preferred_element_type=jnp.float32,
        )
        col = c * SC_CHUNK + jax.lax.broadcasted_iota(
            jnp.int32, (B, SC_CHUNK), 1)
        s_ref[...] = jnp.where(col < S, s, -1e30)


def _qs_call(x, W_key, b_key2d, cube_keys):
    return pl.pallas_call(
        _qs_body,
        grid=(NLTQ + NSC,),
        in_specs=[
            pl.BlockSpec((B, LTQ, D), lambda j: (0, jnp.minimum(j, NLTQ - 1), 0)),
            pl.BlockSpec((D, KD), lambda j: (0, 0)),
            pl.BlockSpec((1, KD), lambda j: (0, 0)),
            pl.BlockSpec((SC_CHUNK, KD),
                         lambda j: (jnp.maximum(j - NLTQ, 0), 0)),
        ],
        out_specs=pl.BlockSpec((B, SC_CHUNK),
                               lambda j: (0, jnp.maximum(j - NLTQ, 0))),
        out_shape=jax.ShapeDtypeStruct((B, S_PAD), jnp.float32),
        scratch_shapes=[
            pltpu.VMEM((B, D), jnp.float32),
            pltpu.VMEM((8, KD), jnp.float32),
        ],
    )(x, W_key, b_key2d, cube_keys)


def _topk_body(s_ref, tv_ref, ti_ref):
    s = s_ref[0]
    idx = (
        jax.lax.broadcasted_iota(jnp.int32, (SROWS, 128), 0) * 128
        + jax.lax.broadcasted_iota(jnp.int32, (SROWS, 128), 1)
    )
    lane = jax.lax.broadcasted_iota(jnp.int32, (1, 1, 128), 2)
    tv = jnp.zeros((1, 1, 128), jnp.float32)
    ti = jnp.zeros((1, 1, 128), jnp.int32)
    for k in range(TOPK):
        m = jnp.max(s)
        cand = jnp.where(s == m, idx, jnp.int32(2**31 - 1))
        fi = jnp.min(cand)
        tv = jnp.where(lane == k, m, tv)
        ti = jnp.where(lane == k, fi, ti)
        s = jnp.where(idx == fi, -3e38, s)
    tv_ref[...] = tv
    ti_ref[...] = ti


def _topk_call(sims3d):
    return pl.pallas_call(
        _topk_body,
        grid=(B,),
        in_specs=[pl.BlockSpec((1, SROWS, 128), lambda b: (b, 0, 0))],
        out_specs=[
            pl.BlockSpec((1, 1, 128), lambda b: (b, 0, 0)),
            pl.BlockSpec((1, 1, 128), lambda b: (b, 0, 0)),
        ],
        out_shape=[
            jax.ShapeDtypeStruct((B, 1, 128), jnp.float32),
            jax.ShapeDtypeStruct((B, 1, 128), jnp.int32),
        ],
    )(sims3d)


def _fused_body(x_ref, wg1_ref, bg1_ref, wrow_ref, conf_ref, mv_ref,
                wmem_ref, bmem_ref, wg2_ref, bg2_ref, lng_ref, lnb_ref,
                out_ref, mem_ref):
    b = pl.program_id(0)
    j = pl.program_id(1)

    @pl.when(jnp.logical_and(b == 0, j == 0))
    def _():
        mem_ref[0:B] = (
            jnp.dot(mv_ref[...], wmem_ref[...],
                    preferred_element_type=jnp.float32)
            + bmem_ref[...]
        )

    xt = x_ref[0]
    t = jnp.dot(xt.astype(jnp.bfloat16), wg1_ref[...],
                preferred_element_type=jnp.float32)
    tb = t + bg1_ref[...] + conf_ref[0, 0] * wrow_ref[...]
    h = 0.5 * tb * (1.0 + jax.lax.erf(tb * 0.7071067811865476))
    sv = jnp.dot(h, wg2_ref[...], preferred_element_type=jnp.float32)
    alpha = jax.nn.sigmoid(sv[:, 0:1] + bg2_ref[0, 0])
    y = xt + (1.0 - alpha) * mem_ref[pl.ds(b, 1)]
    mu = jnp.mean(y, axis=1, keepdims=True)
    var = jnp.mean((y - mu) ** 2, axis=1, keepdims=True)
    out_ref[0] = (y - mu) * jax.lax.rsqrt(var + 1e-5) * lng_ref[...] + lnb_ref[...]


def _fused_call(x, wg1a, bg1, wrow, conf2d, mem_val, W_mem, bmem2d,
                wg2p, bg2v, lng, lnb):
    zero2 = lambda b, j: (0, 0)
    return pl.pallas_call(
        _fused_body,
        grid=(B, NLT),
        in_specs=[
            pl.BlockSpec((1, LT, D), lambda b, j: (b, j, 0)),
            pl.BlockSpec((D, H), zero2),
            pl.BlockSpec((1, H), zero2),
            pl.BlockSpec((1, H), zero2),
            pl.BlockSpec((1, 1), zero2),
            pl.BlockSpec((B, VD), zero2),
            pl.BlockSpec((VD, D), zero2),
            pl.BlockSpec((1, D), zero2),
            pl.BlockSpec((H, 128), zero2),
            pl.BlockSpec((1, 1), zero2),
            pl.BlockSpec((1, D), zero2),
            pl.BlockSpec((1, D), zero2),
        ],
        out_specs=pl.BlockSpec((1, LT, D), lambda b, j: (b, j, 0)),
        out_shape=jax.ShapeDtypeStruct((B, L, D), jnp.float32),
        scratch_shapes=[pltpu.VMEM((8, D), jnp.float32)],
    )(x, wg1a, bg1, wrow, conf2d, mem_val, W_mem, bmem2d, wg2p, bg2v,
      lng, lnb)


def kernel(x, W_key, b_key, cube_keys, cube_values, W_mem, b_mem,
           Wg1, bg1, Wg2, bg2, ln_g, ln_b):
    sims = _qs_call(x, W_key, b_key.reshape(1, KD), cube_keys)
    tv, ti = _topk_call(sims.reshape(B, SROWS, 128))
    topv = tv[:, 0, :TOPK]
    topi = ti[:, 0, :TOPK]
    w = jax.nn.softmax(topv, axis=-1)
    conf = jnp.mean(jnp.max(w, axis=-1))
    gathered = jnp.take(cube_values, topi, axis=0)
    mem_val = jnp.sum(w[..., None] * gathered, axis=1)
    return _fused_call(
        x, Wg1[:D].astype(jnp.bfloat16), bg1.reshape(1, H),
        Wg1[D].reshape(1, H), conf.reshape(1, 1), mem_val, W_mem,
        b_mem.reshape(1, D), jnp.pad(Wg2, ((0, 0), (0, 127))),
        bg2.reshape(1, 1), ln_g.reshape(1, D), ln_b.reshape(1, D))
